# baseline (speedup 1.0000x reference)
import jax
import jax.numpy as jnp
from jax import lax
from jax.experimental import pallas as pl
from jax.experimental.pallas import tpu as pltpu
from jax.experimental.pallas import tpu_sc as plsc

B = 16384
F = 26
V = 100000
NC = 2
NS = 16
NW = NC * NS
BPW = B // NW
IPW = F * BPW


def _sc_body(x_hbm, inv_hbm, w_hbm, bias_hbm, out_hbm,
             idx_v, inv_v, vals_v, valsT_v, out_v, bias_v, spm, sem):
    sid = lax.axis_index("s")
    wid = sid * NC + lax.axis_index("c")
    pltpu.sync_copy(x_hbm.at[wid], idx_v)
    pltpu.sync_copy(inv_hbm, inv_v)
    pltpu.sync_copy(bias_hbm, bias_v)

    pltpu.async_copy(w_hbm.at[idx_v], vals_v, sem).wait()

    pltpu.sync_copy(vals_v, spm.at[pl.ds(sid * IPW, IPW)])
    pltpu.async_copy(spm.at[pl.ds(sid * IPW, IPW)].at[inv_v], valsT_v, sem).wait()

    bias_vec = bias_v[...]
    for c in range(BPW // 16):
        acc = bias_vec
        for f in range(F):
            acc = acc + valsT_v[pl.ds(f * BPW + c * 16, 16)]
        out_v[pl.ds(c * 16, 16)] = acc

    pltpu.sync_copy(out_v, out_hbm.at[wid])


def kernel(x_sparse, W, bias):
    xf = x_sparse.astype(jnp.int32) + jnp.arange(F, dtype=jnp.int32) * V
    x2 = xf.reshape(NW, IPW)
    q = jnp.arange(IPW, dtype=jnp.int32)
    inv = (q % BPW) * F + q // BPW
    wflat = W.reshape(-1)
    bias16 = jnp.broadcast_to(bias.astype(jnp.float32), (16,))
    mesh = plsc.VectorSubcoreMesh(core_axis_name="c", subcore_axis_name="s")
    out = pl.kernel(
        _sc_body,
        out_type=jax.ShapeDtypeStruct((NW, BPW), jnp.float32),
        mesh=mesh,
        scratch_types=[
            pltpu.VMEM((IPW,), jnp.int32),
            pltpu.VMEM((IPW,), jnp.int32),
            pltpu.VMEM((IPW,), jnp.float32),
            pltpu.VMEM((IPW,), jnp.float32),
            pltpu.VMEM((BPW,), jnp.float32),
            pltpu.VMEM((16,), jnp.float32),
            pltpu.VMEM_SHARED((NS * IPW,), jnp.float32),
            pltpu.SemaphoreType.DMA,
        ],
    )(x2, inv, wflat, bias16)
    return out.reshape(B, 1)


# speedup vs baseline: 1.1752x; 1.1752x over previous
import jax
import jax.numpy as jnp
from jax import lax
from jax.experimental import pallas as pl
from jax.experimental.pallas import tpu as pltpu
from jax.experimental.pallas import tpu_sc as plsc

B = 16384
F = 26
V = 100000
NC = 2
NS = 16
NW = NC * NS
NH = 2
BH = B // NH
BPW = BH // NW
IPW = F * BPW


def _sc_body(x_hbm, w_hbm, bias_hbm, out_hbm, idx_v, vals_v, out_v, bias_v, sem):
    wid = lax.axis_index("s") * NC + lax.axis_index("c")
    pltpu.sync_copy(x_hbm.at[wid], idx_v)
    pltpu.sync_copy(bias_hbm, bias_v)

    pltpu.async_copy(w_hbm.at[idx_v], vals_v, sem).wait()

    bias_vec = bias_v[...]
    for c in range(BPW // 16):
        acc = bias_vec
        for f in range(F):
            acc = acc + vals_v[pl.ds(f * BPW + c * 16, 16)]
        out_v[pl.ds(c * 16, 16)] = acc

    pltpu.sync_copy(out_v, out_hbm.at[wid])


def kernel(x_sparse, W, bias):
    xf = x_sparse.astype(jnp.int32) + jnp.arange(F, dtype=jnp.int32) * V
    wflat = W.reshape(-1)
    bias16 = jnp.broadcast_to(bias.astype(jnp.float32), (16,))
    mesh = plsc.VectorSubcoreMesh(core_axis_name="c", subcore_axis_name="s")
    call = pl.kernel(
        _sc_body,
        out_type=jax.ShapeDtypeStruct((NW, BPW), jnp.float32),
        mesh=mesh,
        scratch_types=[
            pltpu.VMEM((IPW,), jnp.int32),
            pltpu.VMEM((IPW,), jnp.float32),
            pltpu.VMEM((BPW,), jnp.float32),
            pltpu.VMEM((16,), jnp.float32),
            pltpu.SemaphoreType.DMA,
        ],
    )
    outs = []
    for h in range(NH):
        xh = xf[h * BH:(h + 1) * BH]
        x2 = xh.T.reshape(F, NW, BPW).transpose(1, 0, 2).reshape(NW, IPW)
        outs.append(call(x2, wflat, bias16).reshape(BH))
    return jnp.concatenate(outs).reshape(B, 1)
